# Initial kernel scaffold; baseline (speedup 1.0000x reference)
import jax
import jax.numpy as jnp
from jax.experimental import pallas as pl


def kernel(nodes, edge_index, W_emb, b_emb, ln_scale, ln_bias, Wq, Wk, Wv, Wo, Wg, bg):
    raise NotImplementedError("write your pallas kernel here")


# speedup vs baseline: 33.6761x; 33.6761x over previous
import functools
import math

import jax
import jax.numpy as jnp
import numpy as np
from jax import lax
from jax.experimental import pallas as pl
from jax.experimental.pallas import tpu as pltpu
from jax.experimental.pallas import tpu_sc as plsc

ALPHA = 0.9
H = 4
DH = 8
CH = 64
ROW = 40
NTILES = 32
LN_EPS = 1e-5
DEN_EPS = 1e-9


def _embed_body(n_ref, w_ref, b_ref, o_ref):
    o_ref[...] = (
        jnp.dot(n_ref[...], w_ref[...], preferred_element_type=jnp.float32)
        + b_ref[...]
    )


def _embed(nodes, W_emb, b_emb2):
    N, D_IN = nodes.shape
    D = W_emb.shape[1]
    BN = 1024
    grid = (pl.cdiv(N, BN),)
    return pl.pallas_call(
        _embed_body,
        grid=grid,
        in_specs=[
            pl.BlockSpec((BN, D_IN), lambda i: (i, 0)),
            pl.BlockSpec((D_IN, D), lambda i: (0, 0)),
            pl.BlockSpec((1, D), lambda i: (0, 0)),
        ],
        out_specs=pl.BlockSpec((BN, D), lambda i: (i, 0)),
        out_shape=jax.ShapeDtypeStruct((N, D), jnp.float32),
    )(nodes, W_emb, b_emb2)


def _qkv_body(x_ref, s_ref, b_ref, w_ref, u_ref):
    x = x_ref[...]
    m = jnp.mean(x, axis=-1, keepdims=True)
    xc = x - m
    v = jnp.mean(xc * xc, axis=-1, keepdims=True)
    h = xc / jnp.sqrt(v + LN_EPS) * s_ref[...] + b_ref[...]
    u_ref[...] = jnp.dot(h, w_ref[...], preferred_element_type=jnp.float32)


def _qkv(x, scale2, bias2, Wqkv):
    N, D = x.shape
    WU = Wqkv.shape[1]
    BN = 2048
    grid = (pl.cdiv(N, BN),)
    return pl.pallas_call(
        _qkv_body,
        grid=grid,
        in_specs=[
            pl.BlockSpec((BN, D), lambda i: (i, 0)),
            pl.BlockSpec((1, D), lambda i: (0, 0)),
            pl.BlockSpec((1, D), lambda i: (0, 0)),
            pl.BlockSpec((D, WU), lambda i: (0, 0)),
        ],
        out_specs=pl.BlockSpec((BN, WU), lambda i: (i, 0)),
        out_shape=jax.ShapeDtypeStruct((N, WU), jnp.float32),
    )(x, scale2, bias2, Wqkv)


def _edgeg_body(NC, idx_hbm, u_hbm, ort_hbm, idx_v, grows, orows, semg):
    c = lax.axis_index("c")
    s = lax.axis_index("s")
    wid = c * 16 + s
    nit = (NC - wid + NTILES - 1) // NTILES
    inv_sqrt = 1.0 / math.sqrt(DH)

    def iter_body(i, carry):
        chunk = wid + i * NTILES
        pltpu.sync_copy(idx_hbm.at[chunk], idx_v)
        pltpu.async_copy(u_hbm.at[idx_v], grows, semg).wait()
        for b in range(CH // 16):
            erows = lax.iota(jnp.int32, 16) + (b * 16)
            krows = erows + CH
            obase = lax.iota(jnp.int32, 16) * ROW + (b * 16 * ROW)
            for h in range(H):
                acc = jnp.zeros((16,), jnp.float32)
                for j in range(DH):
                    qv = plsc.load_gather(
                        grows, [erows, jnp.full((16,), h * DH + j, jnp.int32)])
                    kv = plsc.load_gather(
                        grows, [krows, jnp.full((16,), 32 + h * DH + j,
                                                jnp.int32)])
                    acc = acc + qv * kv
                ex = jnp.exp(acc * inv_sqrt)
                plsc.store_scatter(orows, [obase + (32 + h)], ex)
                if h == 0:
                    zero16 = jnp.zeros((16,), jnp.float32)
                    for pc in range(36, ROW):
                        plsc.store_scatter(orows, [obase + pc], zero16)
                for j in range(DH):
                    vv = plsc.load_gather(
                        grows, [krows, jnp.full((16,), 64 + h * DH + j,
                                                jnp.int32)])
                    plsc.store_scatter(orows, [obase + (h * DH + j)], vv * ex)
        pltpu.sync_copy(orows, ort_hbm.at[chunk])
        return carry

    lax.fori_loop(0, nit, iter_body, 0)


def _edge_gather(idx_packed, U):
    NC = idx_packed.shape[0]
    mesh = plsc.VectorSubcoreMesh(core_axis_name="c", subcore_axis_name="s")
    fn = pl.kernel(
        functools.partial(_edgeg_body, NC),
        out_type=jax.ShapeDtypeStruct((NC, CH * ROW), jnp.float32),
        mesh=mesh,
        scratch_types=[
            pltpu.VMEM((2 * CH,), jnp.int32),
            pltpu.VMEM((2 * CH, 128), jnp.float32),
            pltpu.VMEM((CH * ROW,), jnp.float32),
            pltpu.SemaphoreType.DMA,
        ],
        compiler_params=pltpu.CompilerParams(
            needs_layout_passes=False, use_tc_tiling_on_sc=False),
    )
    return fn(idx_packed, U)


def _edges_body(NC, NP, dst_hbm, ort_hbm, z_hbm, out_hbm, dst_v, rbuf, accum):
    c = lax.axis_index("c")
    s = lax.axis_index("s")
    wid = c * 16 + s
    rows_per_tile = NP // 16
    r0 = s * rows_per_tile
    nit = (NC - wid + NTILES - 1) // NTILES

    pltpu.sync_copy(z_hbm.at[pl.ds(r0, rows_per_tile)],
                    accum.at[pl.ds(r0, rows_per_tile)])
    plsc.subcore_barrier()

    def iter_body(i, carry):
        chunk = wid + i * NTILES
        pltpu.sync_copy(dst_hbm.at[chunk], dst_v)
        pltpu.sync_copy(ort_hbm.at[chunk], rbuf)
        pltpu.sync_copy(rbuf, accum.at[dst_v.at[0]], add=True)
        return carry

    lax.fori_loop(0, nit, iter_body, 0)
    plsc.subcore_barrier()
    pltpu.sync_copy(accum.at[pl.ds(r0, rows_per_tile)],
                    out_hbm.at[c, pl.ds(r0, rows_per_tile)])


def _edge_scatter(dst_chunks, ort, z_hbm):
    NC = dst_chunks.shape[0]
    NP = z_hbm.shape[0]
    mesh = plsc.VectorSubcoreMesh(core_axis_name="c", subcore_axis_name="s")
    fn = pl.kernel(
        functools.partial(_edges_body, NC, NP),
        out_type=jax.ShapeDtypeStruct((2, NP, ROW), jnp.float32),
        mesh=mesh,
        scratch_types=[
            pltpu.VMEM((1, CH), jnp.int32),
            pltpu.VMEM((CH, ROW), jnp.float32),
            pltpu.VMEM_SHARED((NP, ROW), jnp.float32),
        ],
        compiler_params=pltpu.CompilerParams(
            needs_layout_passes=False, use_tc_tiling_on_sc=False),
    )
    return fn(dst_chunks, ort.reshape(NC, CH, ROW), z_hbm)


def _update_body(acc_ref, x_ref, x0_ref, sel_ref, wo_ref, wga_ref, wgb_ref,
                 bg_ref, o_ref):
    a0 = acc_ref[0]
    a1 = acc_ref[1]
    num = a0[:, :32] + a1[:, :32]
    den = a0[:, 32:36] + a1[:, 32:36]
    rec = 1.0 / (den + DEN_EPS)
    rec32 = jnp.dot(rec, sel_ref[...], preferred_element_type=jnp.float32)
    agg = num * rec32
    out = jnp.dot(agg, wo_ref[...], preferred_element_type=jnp.float32)
    x = x_ref[...]
    g = jax.nn.sigmoid(
        jnp.dot(x, wga_ref[...], preferred_element_type=jnp.float32)
        + jnp.dot(out, wgb_ref[...], preferred_element_type=jnp.float32)
        + bg_ref[...]
    )
    o_ref[...] = ALPHA * (x * g + out * (1.0 - g)) + (1.0 - ALPHA) * x0_ref[...]


def _update(accum, x, x0, sel, Wo_l, WgA, WgB, bg2):
    N, D = x.shape
    BN = 2048
    grid = (pl.cdiv(N, BN),)
    return pl.pallas_call(
        _update_body,
        grid=grid,
        in_specs=[
            pl.BlockSpec((2, BN, ROW), lambda i: (0, i, 0)),
            pl.BlockSpec((BN, D), lambda i: (i, 0)),
            pl.BlockSpec((BN, D), lambda i: (i, 0)),
            pl.BlockSpec((4, 32), lambda i: (0, 0)),
            pl.BlockSpec((32, D), lambda i: (0, 0)),
            pl.BlockSpec((D, D), lambda i: (0, 0)),
            pl.BlockSpec((D, D), lambda i: (0, 0)),
            pl.BlockSpec((1, D), lambda i: (0, 0)),
        ],
        out_specs=pl.BlockSpec((BN, D), lambda i: (i, 0)),
        out_shape=jax.ShapeDtypeStruct((N, D), jnp.float32),
    )(accum, x, x0, sel, Wo_l, WgA, WgB, bg2)


def kernel(nodes, edge_index, W_emb, b_emb, ln_scale, ln_bias, Wq, Wk, Wv,
           Wo, Wg, bg):
    N = nodes.shape[0]
    E = edge_index.shape[1]
    D = W_emb.shape[1]
    depth = ln_scale.shape[0]
    NP = ((N + 127) // 128) * 128
    NC = E // CH
    assert NC * CH == E

    src_c = edge_index[0].reshape(NC, CH)
    dst_c = edge_index[1].reshape(NC, CH)
    idx_packed = jnp.concatenate([dst_c, src_c], axis=1)
    dst_chunks = dst_c.reshape(NC, 1, CH)

    x = _embed(nodes, W_emb, b_emb.reshape(1, D))
    x0 = x
    zeros_np = jnp.zeros((NP, ROW), jnp.float32)
    sel = jnp.asarray(np.kron(np.eye(H), np.ones((1, DH))), jnp.float32)
    for l in range(depth):
        Wqkv = jnp.concatenate(
            [Wq[l], Wk[l], Wv[l], jnp.zeros_like(Wq[l])], axis=1)
        WgA = Wg[l, :D] + Wg[l, 2 * D:]
        WgB = Wg[l, D:2 * D] - Wg[l, 2 * D:]
        U = _qkv(x, ln_scale[l].reshape(1, D), ln_bias[l].reshape(1, D), Wqkv)
        ort = _edge_gather(idx_packed, U)
        accum = _edge_scatter(dst_chunks, ort, zeros_np)
        x = _update(accum, x, x0, sel, Wo[l], WgA, WgB, bg[l].reshape(1, D))
    return x
